# baseline (speedup 1.0000x reference)
import functools

import jax
import jax.numpy as jnp
from jax import lax
from jax.experimental import pallas as pl
from jax.experimental.pallas import tpu as pltpu
from jax.experimental.pallas import tpu_sc as plsc

SEQ = 200
BATCH = 4096
HIDDEN = 64
WIDE = 2 * HIDDEN
VOCAB = 1000000
TOT = SEQ * BATCH
CHUNK = 128
NC = 2
NS = 16
NW = NC * NS
CPW = SEQ
NBUF = 4
LOOK = 2


def _gather_body(idx_hbm, t2_hbm, out_hbm, idx_v, *rest):
    gbufs = rest[:NBUF]
    cbufs = rest[NBUF:NBUF + 2]
    sems = rest[NBUF + 2:2 * NBUF + 2]
    stsems = rest[2 * NBUF + 2:]
    wid = lax.axis_index("s") * NC + lax.axis_index("c")
    col0 = wid * CHUNK

    def out_at(c):
        return out_hbm.at[pl.ds(c * BATCH + col0, CHUNK)]

    def gather(c, b):
        pltpu.make_async_copy(t2_hbm.at[idx_v.at[c]], gbufs[b], sems[b]).start()

    def store(c, cb):
        return pltpu.make_async_copy(cbufs[cb], out_at(c), stsems[cb])

    def compact(b, cb):
        gb, ob = gbufs[b], cbufs[cb]

        def rows(r4, carry):
            for rr in range(4):
                r = r4 * 4 + rr
                for j in range(4):
                    ob[r, pl.ds(j * 16, 16)] = gb[r, pl.ds(j * 16, 16)]
            return carry

        lax.fori_loop(0, CHUNK // 4, rows, 0)

    pltpu.sync_copy(idx_hbm.at[:, pl.ds(col0, CHUNK)], idx_v)

    for c in range(LOOK):
        gather(c, c % NBUF)

    def group(g, carry):
        for b in range(NBUF):
            c = g * NBUF + b
            pb = (b + LOOK) % NBUF
            cb = b % 2

            @pl.when(c + LOOK < CPW)
            def _():
                gather(c + LOOK, pb)

            pltpu.make_async_copy(t2_hbm.at[idx_v.at[c]], gbufs[b], sems[b]).wait()

            @pl.when(c >= 2)
            def _():
                store(c - 2, cb).wait()

            compact(b, cb)
            store(c, cb).start()
        return carry

    lax.fori_loop(0, CPW // NBUF, group, 0)

    for c in range(CPW - 2, CPW):
        store(c, c % 2).wait()


def kernel(input_ids, table):
    mesh = plsc.VectorSubcoreMesh(core_axis_name="c", subcore_axis_name="s")
    t2 = jnp.concatenate([table, jnp.zeros((VOCAB, WIDE - HIDDEN), jnp.float32)], axis=1)
    run = functools.partial(
        pl.kernel,
        mesh=mesh,
        out_type=jax.ShapeDtypeStruct((TOT, HIDDEN), jnp.float32),
        scratch_types=[pltpu.VMEM((CPW, CHUNK), jnp.int32)]
        + [pltpu.VMEM((CHUNK, WIDE), jnp.float32) for _ in range(NBUF)]
        + [pltpu.VMEM((CHUNK, HIDDEN), jnp.float32) for _ in range(2)]
        + [pltpu.SemaphoreType.DMA for _ in range(NBUF + 2)],
    )(_gather_body)
    out = run(input_ids.astype(jnp.int32), t2)
    return out.reshape(SEQ, BATCH, HIDDEN)


# speedup vs baseline: 1.0028x; 1.0028x over previous
import functools

import jax
import jax.numpy as jnp
from jax import lax
from jax.experimental import pallas as pl
from jax.experimental.pallas import tpu as pltpu
from jax.experimental.pallas import tpu_sc as plsc

SEQ = 200
BATCH = 4096
HIDDEN = 64
WIDE = 2 * HIDDEN
VOCAB = 1000000
TOT = SEQ * BATCH
CHUNK = 128
NC = 2
NS = 16
NW = NC * NS
CPW = SEQ
NBUF = 4
LOOK = 3


def _gather_body(idx_hbm, t2_hbm, out_hbm, idx_v, *rest):
    gbufs = rest[:NBUF]
    cbufs = rest[NBUF:NBUF + 2]
    sems = rest[NBUF + 2:2 * NBUF + 2]
    stsems = rest[2 * NBUF + 2:]
    wid = lax.axis_index("s") * NC + lax.axis_index("c")
    col0 = wid * CHUNK

    def out_at(c):
        return out_hbm.at[pl.ds(c * BATCH + col0, CHUNK)]

    def gather(c, b):
        pltpu.make_async_copy(t2_hbm.at[idx_v.at[c]], gbufs[b], sems[b]).start()

    def store(c, cb):
        return pltpu.make_async_copy(cbufs[cb], out_at(c), stsems[cb])

    def compact(b, cb):
        gb, ob = gbufs[b], cbufs[cb]

        def rows(r8, carry):
            for rr in range(8):
                r = r8 * 8 + rr
                for j in range(4):
                    ob[r, pl.ds(j * 16, 16)] = gb[r, pl.ds(j * 16, 16)]
            return carry

        lax.fori_loop(0, CHUNK // 8, rows, 0)

    pltpu.sync_copy(idx_hbm.at[:, pl.ds(col0, CHUNK)], idx_v)

    for c in range(LOOK):
        gather(c, c % NBUF)

    def group(g, carry):
        for b in range(NBUF):
            c = g * NBUF + b
            pb = (b + LOOK) % NBUF
            cb = b % 2

            @pl.when(c + LOOK < CPW)
            def _():
                gather(c + LOOK, pb)

            pltpu.make_async_copy(t2_hbm.at[idx_v.at[c]], gbufs[b], sems[b]).wait()

            @pl.when(c >= 2)
            def _():
                store(c - 2, cb).wait()

            compact(b, cb)
            store(c, cb).start()
        return carry

    lax.fori_loop(0, CPW // NBUF, group, 0)

    for c in range(CPW - 2, CPW):
        store(c, c % 2).wait()


def kernel(input_ids, table):
    mesh = plsc.VectorSubcoreMesh(core_axis_name="c", subcore_axis_name="s")
    t2 = jnp.pad(table, ((0, 0), (0, WIDE - HIDDEN)))
    run = functools.partial(
        pl.kernel,
        mesh=mesh,
        out_type=jax.ShapeDtypeStruct((TOT, HIDDEN), jnp.float32),
        scratch_types=[pltpu.VMEM((CPW, CHUNK), jnp.int32)]
        + [pltpu.VMEM((CHUNK, WIDE), jnp.float32) for _ in range(NBUF)]
        + [pltpu.VMEM((CHUNK, HIDDEN), jnp.float32) for _ in range(2)]
        + [pltpu.SemaphoreType.DMA for _ in range(NBUF + 2)],
    )(_gather_body)
    out = run(input_ids.astype(jnp.int32), t2)
    return out.reshape(SEQ, BATCH, HIDDEN)
